# baseline (speedup 1.0000x reference)
import functools

import jax
import jax.numpy as jnp
from jax import lax
from jax.experimental import pallas as pl
from jax.experimental.pallas import tpu as pltpu
from jax.experimental.pallas import tpu_sc as plsc

B = 1024
D = 128
N = 100000
TEMP_INV = 20.0
EPS = 0.1
NBLK = 2000
GRID = N // NBLK
NEG = -1e30


def _gather_rows_sc(features, idx):
    info = plsc.get_sparse_core_info()
    nw = info.num_cores * info.num_subcores
    bpw = B // nw
    mesh = plsc.VectorSubcoreMesh(core_axis_name="c", subcore_axis_name="s")

    @functools.partial(
        pl.kernel, mesh=mesh,
        out_type=jax.ShapeDtypeStruct((B, D), jnp.float32),
        scratch_types=[
            pltpu.VMEM((bpw,), jnp.int32),
            pltpu.VMEM((bpw, D), jnp.float32),
            pltpu.SemaphoreType.DMA,
        ],
    )
    def k(table_hbm, idx_hbm, out_hbm, idx_v, rows_v, sem):
        wid = lax.axis_index("s") * info.num_cores + lax.axis_index("c")
        base = wid * bpw
        pltpu.sync_copy(idx_hbm.at[pl.ds(base, bpw)], idx_v)
        pltpu.async_copy(table_hbm.at[idx_v], rows_v, sem).wait()
        pltpu.sync_copy(rows_v, out_hbm.at[pl.ds(base, bpw)])

    return k(features, idx)


def _tc_body(x_ref, f_ref, g_ref, out_ref, m_ref, s_ref, cs_ref):
    i = pl.program_id(0)

    @pl.when(i == 0)
    def _():
        m_ref[...] = jnp.full((B, 1), NEG, jnp.float32)
        s_ref[...] = jnp.zeros((B, 1), jnp.float32)
        cs_ref[...] = jnp.zeros((1, D), jnp.float32)

    x = x_ref[...]
    nrm = jnp.sqrt(jnp.sum(x * x, axis=1, keepdims=True))
    xn = (TEMP_INV / jnp.maximum(nrm, 1e-12)) * x

    f = f_ref[...]
    fb = f.astype(jnp.bfloat16)
    logits = lax.dot_general(
        xn.astype(jnp.bfloat16), fb, (((1,), (1,)), ((), ())),
        preferred_element_type=jnp.float32)
    lb = logits.astype(jnp.bfloat16)

    m_old = m_ref[...]
    bmax = jnp.max(lb, axis=1, keepdims=True).astype(jnp.float32)
    m_new = jnp.maximum(m_old, bmax)
    p = jnp.exp(lb - m_new.astype(jnp.bfloat16))
    ones = jnp.ones((NBLK, 1), jnp.bfloat16)
    s_blk = lax.dot_general(
        p, ones, (((1,), (0,)), ((), ())), preferred_element_type=jnp.float32)
    s_ref[...] = s_ref[...] * jnp.exp(m_old - m_new) + s_blk
    m_ref[...] = m_new
    cs_ref[...] = cs_ref[...] + jnp.sum(f, axis=0, keepdims=True)

    @pl.when(i == GRID - 1)
    def _():
        tl = jnp.sum(xn * g_ref[...], axis=1, keepdims=True)
        ss = jnp.sum(xn * cs_ref[...], axis=1, keepdims=True)
        lse = m_ref[...] + jnp.log(s_ref[...])
        per_row = lse - (1.0 - EPS) * tl - (EPS / N) * ss
        out_ref[0, 0] = jnp.sum(per_row) / B


def _loss_tc(x, features, gathered, interpret=False):
    out = pl.pallas_call(
        _tc_body,
        grid=(GRID,),
        in_specs=[
            pl.BlockSpec((B, D), lambda i: (0, 0)),
            pl.BlockSpec((NBLK, D), lambda i: (i, 0)),
            pl.BlockSpec((B, D), lambda i: (0, 0)),
        ],
        out_specs=pl.BlockSpec(memory_space=pltpu.SMEM),
        out_shape=jax.ShapeDtypeStruct((1, 1), jnp.float32),
        scratch_shapes=[
            pltpu.VMEM((B, 1), jnp.float32),
            pltpu.VMEM((B, 1), jnp.float32),
            pltpu.VMEM((1, D), jnp.float32),
        ],
        compiler_params=pltpu.CompilerParams(
            dimension_semantics=("arbitrary",)),
        interpret=interpret,
    )(x, features, gathered)
    return out[0, 0]


def kernel(inputs, targets, features):
    gathered = _gather_rows_sc(features, targets.astype(jnp.int32))
    return _loss_tc(inputs, features, gathered)


# speedup vs baseline: 1.1712x; 1.1712x over previous
import functools

import jax
import jax.numpy as jnp
from jax import lax
from jax.experimental import pallas as pl
from jax.experimental.pallas import tpu as pltpu
from jax.experimental.pallas import tpu_sc as plsc

B = 1024
D = 128
N = 100000
TEMP_INV = 20.0
EPS = 0.1
NBLK = 4000
GRID = N // NBLK
NEG = -1e30


def _gather_rows_sc(features, idx):
    info = plsc.get_sparse_core_info()
    nw = info.num_cores * info.num_subcores
    bpw = B // nw
    mesh = plsc.VectorSubcoreMesh(core_axis_name="c", subcore_axis_name="s")

    @functools.partial(
        pl.kernel, mesh=mesh,
        out_type=jax.ShapeDtypeStruct((B, D), jnp.float32),
        scratch_types=[
            pltpu.VMEM((bpw,), jnp.int32),
            pltpu.VMEM((bpw, D), jnp.float32),
            pltpu.SemaphoreType.DMA,
        ],
    )
    def k(table_hbm, idx_hbm, out_hbm, idx_v, rows_v, sem):
        wid = lax.axis_index("s") * info.num_cores + lax.axis_index("c")
        base = wid * bpw
        pltpu.sync_copy(idx_hbm.at[pl.ds(base, bpw)], idx_v)
        pltpu.async_copy(table_hbm.at[idx_v], rows_v, sem).wait()
        pltpu.sync_copy(rows_v, out_hbm.at[pl.ds(base, bpw)])

    return k(features, idx)


def _tc_body(x_ref, f_ref, g_ref, out_ref, m_ref, s_ref, cs_ref):
    i = pl.program_id(0)

    @pl.when(i == 0)
    def _():
        m_ref[...] = jnp.full((B, 1), NEG, jnp.float32)
        s_ref[...] = jnp.zeros((B, 1), jnp.float32)
        cs_ref[...] = jnp.zeros((1, D), jnp.float32)

    x = x_ref[...]
    nrm = jnp.sqrt(jnp.sum(x * x, axis=1, keepdims=True))
    xn = (TEMP_INV / jnp.maximum(nrm, 1e-12)) * x

    f = f_ref[...]
    fb = f.astype(jnp.bfloat16)
    logits = lax.dot_general(
        xn.astype(jnp.bfloat16), fb, (((1,), (1,)), ((), ())),
        preferred_element_type=jnp.float32)
    lb = logits.astype(jnp.bfloat16)

    m_old = m_ref[...]
    bmax = jnp.max(lb, axis=1, keepdims=True).astype(jnp.float32)
    m_new = jnp.maximum(m_old, bmax)
    p = jnp.exp(lb - m_new.astype(jnp.bfloat16))
    ones = jnp.ones((NBLK, 1), jnp.bfloat16)
    s_blk = lax.dot_general(
        p, ones, (((1,), (0,)), ((), ())), preferred_element_type=jnp.float32)
    s_ref[...] = s_ref[...] * jnp.exp(m_old - m_new) + s_blk
    m_ref[...] = m_new
    cs_ref[...] = cs_ref[...] + jnp.sum(f, axis=0, keepdims=True)

    @pl.when(i == GRID - 1)
    def _():
        tl = jnp.sum(xn * g_ref[...], axis=1, keepdims=True)
        ss = jnp.sum(xn * cs_ref[...], axis=1, keepdims=True)
        lse = m_ref[...] + jnp.log(s_ref[...])
        per_row = lse - (1.0 - EPS) * tl - (EPS / N) * ss
        out_ref[0, 0] = jnp.sum(per_row) / B


def _loss_tc(x, features, gathered, interpret=False):
    out = pl.pallas_call(
        _tc_body,
        grid=(GRID,),
        in_specs=[
            pl.BlockSpec((B, D), lambda i: (0, 0)),
            pl.BlockSpec((NBLK, D), lambda i: (i, 0)),
            pl.BlockSpec((B, D), lambda i: (0, 0)),
        ],
        out_specs=pl.BlockSpec(memory_space=pltpu.SMEM),
        out_shape=jax.ShapeDtypeStruct((1, 1), jnp.float32),
        scratch_shapes=[
            pltpu.VMEM((B, 1), jnp.float32),
            pltpu.VMEM((B, 1), jnp.float32),
            pltpu.VMEM((1, D), jnp.float32),
        ],
        compiler_params=pltpu.CompilerParams(
            dimension_semantics=("arbitrary",)),
        interpret=interpret,
    )(x, features, gathered)
    return out[0, 0]


def kernel(inputs, targets, features):
    gathered = _gather_rows_sc(features, targets.astype(jnp.int32))
    return _loss_tc(inputs, features, gathered)


# speedup vs baseline: 1.2153x; 1.0376x over previous
import functools

import jax
import jax.numpy as jnp
from jax import lax
from jax.experimental import pallas as pl
from jax.experimental.pallas import tpu as pltpu
from jax.experimental.pallas import tpu_sc as plsc

B = 1024
D = 128
N = 100000
TEMP_INV = 20.0
EPS = 0.1
NBLK = 5000
GRID = N // NBLK
NEG = -1e30


def _gather_rows_sc(features, idx):
    info = plsc.get_sparse_core_info()
    nw = info.num_cores * info.num_subcores
    bpw = B // nw
    mesh = plsc.VectorSubcoreMesh(core_axis_name="c", subcore_axis_name="s")

    @functools.partial(
        pl.kernel, mesh=mesh,
        out_type=jax.ShapeDtypeStruct((B, D), jnp.float32),
        scratch_types=[
            pltpu.VMEM((bpw,), jnp.int32),
            pltpu.VMEM((bpw, D), jnp.float32),
            pltpu.SemaphoreType.DMA,
        ],
    )
    def k(table_hbm, idx_hbm, out_hbm, idx_v, rows_v, sem):
        wid = lax.axis_index("s") * info.num_cores + lax.axis_index("c")
        base = wid * bpw
        pltpu.sync_copy(idx_hbm.at[pl.ds(base, bpw)], idx_v)
        pltpu.async_copy(table_hbm.at[idx_v], rows_v, sem).wait()
        pltpu.sync_copy(rows_v, out_hbm.at[pl.ds(base, bpw)])

    return k(features, idx)


def _tc_body(x_ref, f_ref, g_ref, out_ref, m_ref, s_ref, cs_ref):
    i = pl.program_id(0)

    @pl.when(i == 0)
    def _():
        m_ref[...] = jnp.full((B, 1), NEG, jnp.float32)
        s_ref[...] = jnp.zeros((B, 1), jnp.float32)
        cs_ref[...] = jnp.zeros((1, D), jnp.float32)

    x = x_ref[...]
    nrm = jnp.sqrt(jnp.sum(x * x, axis=1, keepdims=True))
    xn = (TEMP_INV / jnp.maximum(nrm, 1e-12)) * x

    f = f_ref[...]
    fb = f.astype(jnp.bfloat16)
    logits = lax.dot_general(
        xn.astype(jnp.bfloat16), fb, (((1,), (1,)), ((), ())),
        preferred_element_type=jnp.float32)
    lb = logits.astype(jnp.bfloat16)

    m_old = m_ref[...]
    bmax = jnp.max(lb, axis=1, keepdims=True).astype(jnp.float32)
    m_new = jnp.maximum(m_old, bmax)
    p = jnp.exp(lb - m_new.astype(jnp.bfloat16))
    ones = jnp.ones((NBLK, 1), jnp.bfloat16)
    s_blk = lax.dot_general(
        p, ones, (((1,), (0,)), ((), ())), preferred_element_type=jnp.float32)
    s_ref[...] = s_ref[...] * jnp.exp(m_old - m_new) + s_blk
    m_ref[...] = m_new
    cs_ref[...] = cs_ref[...] + jnp.sum(f, axis=0, keepdims=True)

    @pl.when(i == GRID - 1)
    def _():
        tl = jnp.sum(xn * g_ref[...], axis=1, keepdims=True)
        ss = jnp.sum(xn * cs_ref[...], axis=1, keepdims=True)
        lse = m_ref[...] + jnp.log(s_ref[...])
        per_row = lse - (1.0 - EPS) * tl - (EPS / N) * ss
        out_ref[0, 0] = jnp.sum(per_row) / B


def _loss_tc(x, features, gathered, interpret=False):
    out = pl.pallas_call(
        _tc_body,
        grid=(GRID,),
        in_specs=[
            pl.BlockSpec((B, D), lambda i: (0, 0)),
            pl.BlockSpec((NBLK, D), lambda i: (i, 0)),
            pl.BlockSpec((B, D), lambda i: (0, 0)),
        ],
        out_specs=pl.BlockSpec(memory_space=pltpu.SMEM),
        out_shape=jax.ShapeDtypeStruct((1, 1), jnp.float32),
        scratch_shapes=[
            pltpu.VMEM((B, 1), jnp.float32),
            pltpu.VMEM((B, 1), jnp.float32),
            pltpu.VMEM((1, D), jnp.float32),
        ],
        compiler_params=pltpu.CompilerParams(
            dimension_semantics=("arbitrary",)),
        interpret=interpret,
    )(x, features, gathered)
    return out[0, 0]


def kernel(inputs, targets, features):
    gathered = _gather_rows_sc(features, targets.astype(jnp.int32))
    return _loss_tc(inputs, features, gathered)


# speedup vs baseline: 1.2834x; 1.0561x over previous
import functools

import jax
import jax.numpy as jnp
from jax import lax
from jax.experimental import pallas as pl
from jax.experimental.pallas import tpu as pltpu
from jax.experimental.pallas import tpu_sc as plsc

B = 1024
D = 128
N = 100000
TEMP_INV = 20.0
EPS = 0.1
NBLK = 5000
GRID = N // NBLK
NEG = -1e30


def _gather_rows_sc(features, idx):
    info = plsc.get_sparse_core_info()
    nw = info.num_cores * info.num_subcores
    bpw = B // nw
    mesh = plsc.VectorSubcoreMesh(core_axis_name="c", subcore_axis_name="s")

    @functools.partial(
        pl.kernel, mesh=mesh,
        out_type=jax.ShapeDtypeStruct((B, D), jnp.float32),
        scratch_types=[
            pltpu.VMEM((bpw,), jnp.int32),
            pltpu.VMEM((bpw, D), jnp.float32),
            pltpu.SemaphoreType.DMA,
        ],
    )
    def k(table_hbm, idx_hbm, out_hbm, idx_v, rows_v, sem):
        wid = lax.axis_index("s") * info.num_cores + lax.axis_index("c")
        base = wid * bpw
        pltpu.sync_copy(idx_hbm.at[pl.ds(base, bpw)], idx_v)
        pltpu.async_copy(table_hbm.at[idx_v], rows_v, sem).wait()
        pltpu.sync_copy(rows_v, out_hbm.at[pl.ds(base, bpw)])

    return k(features, idx)


def _tc_body(x_ref, f_ref, g_ref, out_ref, m_ref, s_ref, cs_ref):
    i = pl.program_id(0)

    @pl.when(i == 0)
    def _():
        m_ref[...] = jnp.full((B, 1), NEG, jnp.float32)
        s_ref[...] = jnp.zeros((B, 1), jnp.float32)
        cs_ref[...] = jnp.zeros((1, D), jnp.float32)

    x = x_ref[...]
    nrm = jnp.sqrt(jnp.sum(x * x, axis=1, keepdims=True))
    xn = (TEMP_INV / jnp.maximum(nrm, 1e-12)) * x

    f = f_ref[...]
    fb = f.astype(jnp.bfloat16)
    logits = lax.dot_general(
        xn.astype(jnp.bfloat16), fb, (((1,), (1,)), ((), ())),
        preferred_element_type=jnp.float32)
    lb = logits.astype(jnp.bfloat16)

    m_old = m_ref[...]
    bmax = jnp.max(lb, axis=1, keepdims=True).astype(jnp.float32)
    m_new = jnp.maximum(m_old, bmax)
    s_blk = jnp.sum(jnp.exp(logits - m_new), axis=1, keepdims=True)
    s_ref[...] = s_ref[...] * jnp.exp(m_old - m_new) + s_blk
    m_ref[...] = m_new
    cs_ref[...] = cs_ref[...] + jnp.sum(f, axis=0, keepdims=True)

    @pl.when(i == GRID - 1)
    def _():
        tl = jnp.sum(xn * g_ref[...], axis=1, keepdims=True)
        ss = jnp.sum(xn * cs_ref[...], axis=1, keepdims=True)
        lse = m_ref[...] + jnp.log(s_ref[...])
        per_row = lse - (1.0 - EPS) * tl - (EPS / N) * ss
        out_ref[0, 0] = jnp.sum(per_row) / B


def _loss_tc(x, features, gathered, interpret=False):
    out = pl.pallas_call(
        _tc_body,
        grid=(GRID,),
        in_specs=[
            pl.BlockSpec((B, D), lambda i: (0, 0)),
            pl.BlockSpec((NBLK, D), lambda i: (i, 0)),
            pl.BlockSpec((B, D), lambda i: (0, 0)),
        ],
        out_specs=pl.BlockSpec(memory_space=pltpu.SMEM),
        out_shape=jax.ShapeDtypeStruct((1, 1), jnp.float32),
        scratch_shapes=[
            pltpu.VMEM((B, 1), jnp.float32),
            pltpu.VMEM((B, 1), jnp.float32),
            pltpu.VMEM((1, D), jnp.float32),
        ],
        compiler_params=pltpu.CompilerParams(
            dimension_semantics=("arbitrary",)),
        interpret=interpret,
    )(x, features, gathered)
    return out[0, 0]


def kernel(inputs, targets, features):
    gathered = _gather_rows_sc(features, targets.astype(jnp.int32))
    return _loss_tc(inputs, features, gathered)


# speedup vs baseline: 1.6220x; 1.2639x over previous
import functools

import jax
import jax.numpy as jnp
from jax import lax
from jax.experimental import pallas as pl
from jax.experimental.pallas import tpu as pltpu
from jax.experimental.pallas import tpu_sc as plsc

B = 1024
D = 128
N = 100000
TEMP_INV = 20.0
EPS = 0.1
NBLK = 4096
GRID = (N + NBLK - 1) // NBLK
TAIL = N - (GRID - 1) * NBLK
C_INIT = 110.0
CLAMP_HI = 80.0
DROP_LO = -50.0


def _gather_rows_sc(features, idx):
    info = plsc.get_sparse_core_info()
    nw = info.num_cores * info.num_subcores
    bpw = B // nw
    mesh = plsc.VectorSubcoreMesh(core_axis_name="c", subcore_axis_name="s")

    @functools.partial(
        pl.kernel, mesh=mesh,
        out_type=jax.ShapeDtypeStruct((B, D), jnp.float32),
        scratch_types=[
            pltpu.VMEM((bpw,), jnp.int32),
            pltpu.VMEM((bpw, D), jnp.float32),
            pltpu.SemaphoreType.DMA,
        ],
    )
    def k(table_hbm, idx_hbm, out_hbm, idx_v, rows_v, sem):
        wid = lax.axis_index("s") * info.num_cores + lax.axis_index("c")
        base = wid * bpw
        pltpu.sync_copy(idx_hbm.at[pl.ds(base, bpw)], idx_v)
        pltpu.async_copy(table_hbm.at[idx_v], rows_v, sem).wait()
        pltpu.sync_copy(rows_v, out_hbm.at[pl.ds(base, bpw)])

    return k(features, idx)


def _tc_body(x_ref, f_ref, g_ref, out_ref, m_ref, s_ref, cs_ref):
    i = pl.program_id(0)

    @pl.when(i == 0)
    def _():
        m_ref[...] = jnp.full((B, 1), C_INIT, jnp.float32)
        s_ref[...] = jnp.zeros((B, 1), jnp.float32)
        cs_ref[...] = jnp.zeros((1, D), jnp.float32)

    @pl.when(i == GRID - 1)
    def _():
        f_ref[pl.ds(TAIL, NBLK - TAIL), :] = jnp.zeros(
            (NBLK - TAIL, D), jnp.float32)

    x = x_ref[...]
    nrm = jnp.sqrt(jnp.sum(x * x, axis=1, keepdims=True))
    xn = (TEMP_INV / jnp.maximum(nrm, 1e-12)) * x

    f = f_ref[...]
    fb = f.astype(jnp.bfloat16)
    cb = m_ref[...].astype(jnp.bfloat16)
    c_r = cb.astype(jnp.float32)
    xa = jnp.concatenate([xn.astype(jnp.bfloat16), cb], axis=1)
    fa = jnp.concatenate(
        [fb, jnp.full((NBLK, 1), -1.0, jnp.bfloat16)], axis=1)
    ls = lax.dot_general(
        xa, fa, (((1,), (1,)), ((), ())), preferred_element_type=jnp.float32)

    lsb = ls.astype(jnp.bfloat16)
    bm = jnp.max(lsb, axis=1, keepdims=True).astype(jnp.float32)
    p = jnp.exp(jnp.minimum(lsb, CLAMP_HI))
    q = p[:, :2048] + p[:, 2048:]
    q = q[:, :1024] + q[:, 1024:]
    q = q[:, :512] + q[:, 512:]
    ones = jnp.ones((512, 1), jnp.bfloat16)
    s_blk = lax.dot_general(
        q, ones, (((1,), (0,)), ((), ())), preferred_element_type=jnp.float32)

    delta = jnp.maximum(bm, DROP_LO)
    c_new = (c_r + delta).astype(jnp.bfloat16).astype(jnp.float32)
    s_ref[...] = (s_ref[...] + s_blk) * jnp.exp(c_r - c_new)
    m_ref[...] = c_new
    cs_ref[...] = cs_ref[...] + jnp.sum(f, axis=0, keepdims=True)

    @pl.when(i == GRID - 1)
    def _():
        tl = jnp.sum(xn * g_ref[...], axis=1, keepdims=True)
        ss = jnp.sum(xn * cs_ref[...], axis=1, keepdims=True)
        lse = m_ref[...] + jnp.log(s_ref[...])
        per_row = lse - (1.0 - EPS) * tl - (EPS / N) * ss
        out_ref[0, 0] = jnp.sum(per_row) / B


def _loss_tc(x, features, gathered, interpret=False):
    out = pl.pallas_call(
        _tc_body,
        grid=(GRID,),
        in_specs=[
            pl.BlockSpec((B, D), lambda i: (0, 0)),
            pl.BlockSpec((NBLK, D), lambda i: (i, 0)),
            pl.BlockSpec((B, D), lambda i: (0, 0)),
        ],
        out_specs=pl.BlockSpec(memory_space=pltpu.SMEM),
        out_shape=jax.ShapeDtypeStruct((1, 1), jnp.float32),
        scratch_shapes=[
            pltpu.VMEM((B, 1), jnp.float32),
            pltpu.VMEM((B, 1), jnp.float32),
            pltpu.VMEM((1, D), jnp.float32),
        ],
        compiler_params=pltpu.CompilerParams(
            dimension_semantics=("arbitrary",)),
        interpret=interpret,
    )(x, features, gathered)
    return out[0, 0]


def kernel(inputs, targets, features):
    gathered = _gather_rows_sc(features, targets.astype(jnp.int32))
    return _loss_tc(inputs, features, gathered)


# speedup vs baseline: 1.6285x; 1.0040x over previous
import functools

import jax
import jax.numpy as jnp
from jax import lax
from jax.experimental import pallas as pl
from jax.experimental.pallas import tpu as pltpu
from jax.experimental.pallas import tpu_sc as plsc

B = 1024
D = 128
N = 100000
TEMP_INV = 20.0
EPS = 0.1
NBLK = 4096
GRID = (N + NBLK - 1) // NBLK
TAIL = N - (GRID - 1) * NBLK
C_INIT = 110.0
CLAMP_HI = 80.0
DROP_LO = -50.0


def _gather_rows_sc(features, idx):
    info = plsc.get_sparse_core_info()
    nw = info.num_cores * info.num_subcores
    bpw = B // nw
    mesh = plsc.VectorSubcoreMesh(core_axis_name="c", subcore_axis_name="s")

    @functools.partial(
        pl.kernel, mesh=mesh,
        out_type=jax.ShapeDtypeStruct((B, D), jnp.float32),
        scratch_types=[
            pltpu.VMEM((bpw,), jnp.int32),
            pltpu.VMEM((bpw, D), jnp.float32),
            pltpu.SemaphoreType.DMA,
        ],
    )
    def k(table_hbm, idx_hbm, out_hbm, idx_v, rows_v, sem):
        wid = lax.axis_index("s") * info.num_cores + lax.axis_index("c")
        base = wid * bpw
        pltpu.sync_copy(idx_hbm.at[pl.ds(base, bpw)], idx_v)
        pltpu.async_copy(table_hbm.at[idx_v], rows_v, sem).wait()
        pltpu.sync_copy(rows_v, out_hbm.at[pl.ds(base, bpw)])

    return k(features, idx)


def _tc_body(x_ref, f_ref, g_ref, out_ref, m_ref, s_ref, cs_ref):
    i = pl.program_id(0)

    @pl.when(i == 0)
    def _():
        m_ref[...] = jnp.full((B, 1), C_INIT, jnp.float32)
        s_ref[...] = jnp.zeros((B, 1), jnp.float32)
        cs_ref[...] = jnp.zeros((1, D), jnp.float32)

    @pl.when(i == GRID - 1)
    def _():
        f_ref[pl.ds(TAIL, NBLK - TAIL), :] = jnp.zeros(
            (NBLK - TAIL, D), jnp.float32)

    x = x_ref[...]
    nrm = jnp.sqrt(jnp.sum(x * x, axis=1, keepdims=True))
    xn = (TEMP_INV / jnp.maximum(nrm, 1e-12)) * x

    f = f_ref[...]
    fb = f.astype(jnp.bfloat16)
    cb = m_ref[...].astype(jnp.bfloat16)
    c_r = cb.astype(jnp.float32)
    xa = jnp.concatenate([xn.astype(jnp.bfloat16), cb], axis=1)
    fa = jnp.concatenate(
        [fb, jnp.full((NBLK, 1), -1.0, jnp.bfloat16)], axis=1)
    ls = lax.dot_general(
        xa, fa, (((1,), (1,)), ((), ())), preferred_element_type=jnp.float32)

    lsb = ls.astype(jnp.bfloat16)
    p = jnp.exp(jnp.minimum(lsb, CLAMP_HI))
    q = p[:, :2048] + p[:, 2048:]
    q = q[:, :1024] + q[:, 1024:]
    q = q[:, :512] + q[:, 512:]
    ones = jnp.ones((512, 1), jnp.bfloat16)
    s_blk = lax.dot_general(
        q, ones, (((1,), (0,)), ((), ())), preferred_element_type=jnp.float32)

    delta = jnp.maximum(jnp.log(s_blk), DROP_LO)
    c_new = (c_r + delta).astype(jnp.bfloat16).astype(jnp.float32)
    s_ref[...] = (s_ref[...] + s_blk) * jnp.exp(c_r - c_new)
    m_ref[...] = c_new
    cs_ref[...] = cs_ref[...] + jnp.sum(f, axis=0, keepdims=True)

    @pl.when(i == GRID - 1)
    def _():
        tl = jnp.sum(xn * g_ref[...], axis=1, keepdims=True)
        ss = jnp.sum(xn * cs_ref[...], axis=1, keepdims=True)
        lse = m_ref[...] + jnp.log(s_ref[...])
        per_row = lse - (1.0 - EPS) * tl - (EPS / N) * ss
        out_ref[0, 0] = jnp.sum(per_row) / B


def _loss_tc(x, features, gathered, interpret=False):
    out = pl.pallas_call(
        _tc_body,
        grid=(GRID,),
        in_specs=[
            pl.BlockSpec((B, D), lambda i: (0, 0)),
            pl.BlockSpec((NBLK, D), lambda i: (i, 0)),
            pl.BlockSpec((B, D), lambda i: (0, 0)),
        ],
        out_specs=pl.BlockSpec(memory_space=pltpu.SMEM),
        out_shape=jax.ShapeDtypeStruct((1, 1), jnp.float32),
        scratch_shapes=[
            pltpu.VMEM((B, 1), jnp.float32),
            pltpu.VMEM((B, 1), jnp.float32),
            pltpu.VMEM((1, D), jnp.float32),
        ],
        compiler_params=pltpu.CompilerParams(
            dimension_semantics=("arbitrary",)),
        interpret=interpret,
    )(x, features, gathered)
    return out[0, 0]


def kernel(inputs, targets, features):
    gathered = _gather_rows_sc(features, targets.astype(jnp.int32))
    return _loss_tc(inputs, features, gathered)


# speedup vs baseline: 1.6498x; 1.0131x over previous
import functools

import jax
import jax.numpy as jnp
from jax import lax
from jax.experimental import pallas as pl
from jax.experimental.pallas import tpu as pltpu
from jax.experimental.pallas import tpu_sc as plsc

B = 1024
D = 128
N = 100000
TEMP_INV = 20.0
EPS = 0.1
NBLK = 8192
GRID = (N + NBLK - 1) // NBLK
TAIL = N - (GRID - 1) * NBLK
C_INIT = 110.0
CLAMP_HI = 79.0
DROP_LO = -50.0


def _gather_rows_sc(features, idx):
    info = plsc.get_sparse_core_info()
    nw = info.num_cores * info.num_subcores
    bpw = B // nw
    mesh = plsc.VectorSubcoreMesh(core_axis_name="c", subcore_axis_name="s")

    @functools.partial(
        pl.kernel, mesh=mesh,
        out_type=jax.ShapeDtypeStruct((B, D), jnp.float32),
        scratch_types=[
            pltpu.VMEM((bpw,), jnp.int32),
            pltpu.VMEM((bpw, D), jnp.float32),
            pltpu.SemaphoreType.DMA,
        ],
    )
    def k(table_hbm, idx_hbm, out_hbm, idx_v, rows_v, sem):
        wid = lax.axis_index("s") * info.num_cores + lax.axis_index("c")
        base = wid * bpw
        pltpu.sync_copy(idx_hbm.at[pl.ds(base, bpw)], idx_v)
        pltpu.async_copy(table_hbm.at[idx_v], rows_v, sem).wait()
        pltpu.sync_copy(rows_v, out_hbm.at[pl.ds(base, bpw)])

    return k(features, idx)


def _tc_body(x_ref, f_ref, g_ref, out_ref, m_ref, s_ref, cs_ref):
    i = pl.program_id(0)

    @pl.when(i == 0)
    def _():
        m_ref[...] = jnp.full((B, 1), C_INIT, jnp.float32)
        s_ref[...] = jnp.zeros((B, 1), jnp.float32)
        cs_ref[...] = jnp.zeros((1, D), jnp.float32)

    @pl.when(i == GRID - 1)
    def _():
        f_ref[pl.ds(TAIL, NBLK - TAIL), :] = jnp.zeros(
            (NBLK - TAIL, D), jnp.float32)

    x = x_ref[...]
    nrm = jnp.sqrt(jnp.sum(x * x, axis=1, keepdims=True))
    xn = (TEMP_INV / jnp.maximum(nrm, 1e-12)) * x

    f = f_ref[...]
    fb = f.astype(jnp.bfloat16)
    cb = m_ref[...].astype(jnp.bfloat16)
    c_r = cb.astype(jnp.float32)
    xa = jnp.concatenate([xn.astype(jnp.bfloat16), cb], axis=1)
    fa = jnp.concatenate(
        [fb, jnp.full((NBLK, 1), -1.0, jnp.bfloat16)], axis=1)
    ls = lax.dot_general(
        xa, fa, (((1,), (1,)), ((), ())), preferred_element_type=jnp.float32)

    lsb = ls.astype(jnp.bfloat16)
    p = jnp.exp(jnp.minimum(lsb, CLAMP_HI))
    q = p[:, :4096] + p[:, 4096:]
    q = q[:, :2048] + q[:, 2048:]
    q = q[:, :1024] + q[:, 1024:]
    q = q[:, :512] + q[:, 512:]
    ones = jnp.ones((512, 1), jnp.bfloat16)
    s_blk = lax.dot_general(
        q, ones, (((1,), (0,)), ((), ())), preferred_element_type=jnp.float32)

    delta = jnp.maximum(jnp.log(s_blk), DROP_LO)
    c_new = (c_r + delta).astype(jnp.bfloat16).astype(jnp.float32)
    s_ref[...] = (s_ref[...] + s_blk) * jnp.exp(c_r - c_new)
    m_ref[...] = c_new
    cs_ref[...] = cs_ref[...] + jnp.sum(f, axis=0, keepdims=True)

    @pl.when(i == GRID - 1)
    def _():
        tl = jnp.sum(xn * g_ref[...], axis=1, keepdims=True)
        ss = jnp.sum(xn * cs_ref[...], axis=1, keepdims=True)
        lse = m_ref[...] + jnp.log(s_ref[...])
        per_row = lse - (1.0 - EPS) * tl - (EPS / N) * ss
        out_ref[0, 0] = jnp.sum(per_row) / B


def _loss_tc(x, features, gathered, interpret=False):
    out = pl.pallas_call(
        _tc_body,
        grid=(GRID,),
        in_specs=[
            pl.BlockSpec((B, D), lambda i: (0, 0)),
            pl.BlockSpec((NBLK, D), lambda i: (i, 0)),
            pl.BlockSpec((B, D), lambda i: (0, 0)),
        ],
        out_specs=pl.BlockSpec(memory_space=pltpu.SMEM),
        out_shape=jax.ShapeDtypeStruct((1, 1), jnp.float32),
        scratch_shapes=[
            pltpu.VMEM((B, 1), jnp.float32),
            pltpu.VMEM((B, 1), jnp.float32),
            pltpu.VMEM((1, D), jnp.float32),
        ],
        compiler_params=pltpu.CompilerParams(
            dimension_semantics=("arbitrary",)),
        interpret=interpret,
    )(x, features, gathered)
    return out[0, 0]


def kernel(inputs, targets, features):
    gathered = _gather_rows_sc(features, targets.astype(jnp.int32))
    return _loss_tc(inputs, features, gathered)


# speedup vs baseline: 1.6643x; 1.0088x over previous
import functools

import jax
import jax.numpy as jnp
from jax import lax
from jax.experimental import pallas as pl
from jax.experimental.pallas import tpu as pltpu
from jax.experimental.pallas import tpu_sc as plsc

B = 1024
D = 128
N = 100000
TEMP_INV = 20.0
EPS = 0.1
NBLK = 8192
GRID = (N + NBLK - 1) // NBLK
TAIL = N - (GRID - 1) * NBLK
C_INIT = 110.0
CLAMP_HI = 79.0
DROP_LO = -50.0


def _gather_rows_sc(features, idx):
    info = plsc.get_sparse_core_info()
    nw = info.num_cores * info.num_subcores
    bpw = B // nw
    mesh = plsc.VectorSubcoreMesh(core_axis_name="c", subcore_axis_name="s")

    @functools.partial(
        pl.kernel, mesh=mesh,
        out_type=jax.ShapeDtypeStruct((B, D), jnp.float32),
        scratch_types=[
            pltpu.VMEM((bpw,), jnp.int32),
            pltpu.VMEM((bpw, D), jnp.float32),
            pltpu.SemaphoreType.DMA,
        ],
    )
    def k(table_hbm, idx_hbm, out_hbm, idx_v, rows_v, sem):
        wid = lax.axis_index("s") * info.num_cores + lax.axis_index("c")
        base = wid * bpw
        pltpu.sync_copy(idx_hbm.at[pl.ds(base, bpw)], idx_v)
        pltpu.async_copy(table_hbm.at[idx_v], rows_v, sem).wait()
        pltpu.sync_copy(rows_v, out_hbm.at[pl.ds(base, bpw)])

    return k(features, idx)


def _tc_body(x_ref, f_ref, m_ref, s_ref, cs_ref):
    i = pl.program_id(0)

    @pl.when(i == 0)
    def _():
        m_ref[...] = jnp.full((B, 1), C_INIT, jnp.float32)
        s_ref[...] = jnp.zeros((B, 1), jnp.float32)
        cs_ref[...] = jnp.zeros((1, D), jnp.float32)

    @pl.when(i == GRID - 1)
    def _():
        f_ref[pl.ds(TAIL, NBLK - TAIL), :] = jnp.zeros(
            (NBLK - TAIL, D), jnp.float32)

    x = x_ref[...]
    nrm = jnp.sqrt(jnp.sum(x * x, axis=1, keepdims=True))
    xn = (TEMP_INV / jnp.maximum(nrm, 1e-12)) * x

    f = f_ref[...]
    fb = f.astype(jnp.bfloat16)
    cb = m_ref[...].astype(jnp.bfloat16)
    c_r = cb.astype(jnp.float32)
    xa = jnp.concatenate([xn.astype(jnp.bfloat16), cb], axis=1)
    fa = jnp.concatenate(
        [fb, jnp.full((NBLK, 1), -1.0, jnp.bfloat16)], axis=1)
    ls = lax.dot_general(
        xa, fa, (((1,), (1,)), ((), ())), preferred_element_type=jnp.float32)

    lsb = ls.astype(jnp.bfloat16)
    p = jnp.exp(jnp.minimum(lsb, CLAMP_HI))
    q = p[:, :4096] + p[:, 4096:]
    q = q[:, :2048] + q[:, 2048:]
    q = q[:, :1024] + q[:, 1024:]
    q = q[:, :512] + q[:, 512:]
    ones = jnp.ones((512, 1), jnp.bfloat16)
    s_blk = lax.dot_general(
        q, ones, (((1,), (0,)), ((), ())), preferred_element_type=jnp.float32)

    delta = jnp.maximum(jnp.log(s_blk), DROP_LO)
    c_new = (c_r + delta).astype(jnp.bfloat16).astype(jnp.float32)
    s_ref[...] = (s_ref[...] + s_blk) * jnp.exp(c_r - c_new)
    m_ref[...] = c_new
    cs_ref[...] = cs_ref[...] + jnp.sum(f, axis=0, keepdims=True)


def _flash_tc(x, features, interpret=False):
    return pl.pallas_call(
        _tc_body,
        grid=(GRID,),
        in_specs=[
            pl.BlockSpec((B, D), lambda i: (0, 0)),
            pl.BlockSpec((NBLK, D), lambda i: (i, 0)),
        ],
        out_specs=[
            pl.BlockSpec((B, 1), lambda i: (0, 0)),
            pl.BlockSpec((B, 1), lambda i: (0, 0)),
            pl.BlockSpec((1, D), lambda i: (0, 0)),
        ],
        out_shape=[
            jax.ShapeDtypeStruct((B, 1), jnp.float32),
            jax.ShapeDtypeStruct((B, 1), jnp.float32),
            jax.ShapeDtypeStruct((1, D), jnp.float32),
        ],
        compiler_params=pltpu.CompilerParams(
            dimension_semantics=("arbitrary",)),
        interpret=interpret,
    )(x, features)


def _combine_body(x_ref, g_ref, m_ref, s_ref, cs_ref, out_ref):
    x = x_ref[...]
    nrm = jnp.sqrt(jnp.sum(x * x, axis=1, keepdims=True))
    xn = (TEMP_INV / jnp.maximum(nrm, 1e-12)) * x
    tl = jnp.sum(xn * g_ref[...], axis=1, keepdims=True)
    ss = jnp.sum(xn * cs_ref[...], axis=1, keepdims=True)
    lse = m_ref[...] + jnp.log(s_ref[...])
    per_row = lse - (1.0 - EPS) * tl - (EPS / N) * ss
    out_ref[0, 0] = jnp.sum(per_row) / B


def _combine_tc(x, gathered, m, s, cs, interpret=False):
    out = pl.pallas_call(
        _combine_body,
        out_specs=pl.BlockSpec(memory_space=pltpu.SMEM),
        out_shape=jax.ShapeDtypeStruct((1, 1), jnp.float32),
        interpret=interpret,
    )(x, gathered, m, s, cs)
    return out[0, 0]


def _loss_tc(x, features, gathered, interpret=False):
    m, s, cs = _flash_tc(x, features, interpret=interpret)
    return _combine_tc(x, gathered, m, s, cs, interpret=interpret)


def kernel(inputs, targets, features):
    gathered = _gather_rows_sc(features, targets.astype(jnp.int32))
    m, s, cs = _flash_tc(inputs, features)
    return _combine_tc(inputs, gathered, m, s, cs)


# speedup vs baseline: 1.7707x; 1.0640x over previous
import functools

import jax
import jax.numpy as jnp
from jax import lax
from jax.experimental import pallas as pl
from jax.experimental.pallas import tpu as pltpu
from jax.experimental.pallas import tpu_sc as plsc

B = 1024
D = 128
N = 100000
TEMP_INV = 20.0
EPS = 0.1
NBLK = 8192
GRID = (N + NBLK - 1) // NBLK
TAIL = N - (GRID - 1) * NBLK
C_INIT = 110.0
CLAMP_HI = 79.0
DROP_LO = -50.0


def _gather_rows_sc(features, idx):
    info = plsc.get_sparse_core_info()
    nw = info.num_cores * info.num_subcores
    bpw = B // nw
    mesh = plsc.VectorSubcoreMesh(core_axis_name="c", subcore_axis_name="s")

    @functools.partial(
        pl.kernel, mesh=mesh,
        out_type=jax.ShapeDtypeStruct((B, D), jnp.float32),
        scratch_types=[
            pltpu.VMEM((bpw,), jnp.int32),
            pltpu.VMEM((bpw, D), jnp.float32),
            pltpu.SemaphoreType.DMA,
        ],
    )
    def k(table_hbm, idx_hbm, out_hbm, idx_v, rows_v, sem):
        wid = lax.axis_index("s") * info.num_cores + lax.axis_index("c")
        base = wid * bpw
        pltpu.sync_copy(idx_hbm.at[pl.ds(base, bpw)], idx_v)
        pltpu.async_copy(table_hbm.at[idx_v], rows_v, sem).wait()
        pltpu.sync_copy(rows_v, out_hbm.at[pl.ds(base, bpw)])

    return k(features, idx)


def _tc_body(x_ref, f_ref, m_ref, s_ref, cs_ref):
    i = pl.program_id(0)

    @pl.when(i == 0)
    def _():
        m_ref[...] = jnp.full((B, 1), C_INIT, jnp.float32)
        s_ref[...] = jnp.zeros((B, 1), jnp.float32)
        cs_ref[...] = jnp.zeros((1, D), jnp.float32)

    @pl.when(i == GRID - 1)
    def _():
        f_ref[pl.ds(TAIL, NBLK - TAIL), :] = jnp.zeros(
            (NBLK - TAIL, D), jnp.float32)

    x = x_ref[...]
    nrm = jnp.sqrt(jnp.sum(x * x, axis=1, keepdims=True))
    xn = (TEMP_INV / jnp.maximum(nrm, 1e-12)) * x

    f = f_ref[...]
    fb = f.astype(jnp.bfloat16)
    cb = m_ref[...].astype(jnp.bfloat16)
    c_r = cb.astype(jnp.float32)
    ls = lax.dot_general(
        xn.astype(jnp.bfloat16), fb, (((1,), (1,)), ((), ())),
        preferred_element_type=jnp.float32)

    lsb = ls.astype(jnp.bfloat16)
    p = jnp.exp(jnp.minimum(lsb - cb, CLAMP_HI))
    q = p[:, :4096] + p[:, 4096:]
    q = q[:, :2048] + q[:, 2048:]
    q = q[:, :1024] + q[:, 1024:]
    q = q[:, :512] + q[:, 512:]
    ones = jnp.ones((512, 1), jnp.bfloat16)
    s_blk = lax.dot_general(
        q, ones, (((1,), (0,)), ((), ())), preferred_element_type=jnp.float32)

    delta = jnp.maximum(jnp.log(s_blk), DROP_LO)
    c_new = (c_r + delta).astype(jnp.bfloat16).astype(jnp.float32)
    s_ref[...] = (s_ref[...] + s_blk) * jnp.exp(c_r - c_new)
    m_ref[...] = c_new
    cs_ref[...] = cs_ref[...] + jnp.sum(f, axis=0, keepdims=True)


def _flash_tc(x, features, interpret=False):
    return pl.pallas_call(
        _tc_body,
        grid=(GRID,),
        in_specs=[
            pl.BlockSpec((B, D), lambda i: (0, 0)),
            pl.BlockSpec((NBLK, D), lambda i: (i, 0)),
        ],
        out_specs=[
            pl.BlockSpec((B, 1), lambda i: (0, 0)),
            pl.BlockSpec((B, 1), lambda i: (0, 0)),
            pl.BlockSpec((1, D), lambda i: (0, 0)),
        ],
        out_shape=[
            jax.ShapeDtypeStruct((B, 1), jnp.float32),
            jax.ShapeDtypeStruct((B, 1), jnp.float32),
            jax.ShapeDtypeStruct((1, D), jnp.float32),
        ],
        compiler_params=pltpu.CompilerParams(
            dimension_semantics=("arbitrary",)),
        interpret=interpret,
    )(x, features)


def _combine_body(x_ref, g_ref, m_ref, s_ref, cs_ref, out_ref):
    x = x_ref[...]
    nrm = jnp.sqrt(jnp.sum(x * x, axis=1, keepdims=True))
    xn = (TEMP_INV / jnp.maximum(nrm, 1e-12)) * x
    tl = jnp.sum(xn * g_ref[...], axis=1, keepdims=True)
    ss = jnp.sum(xn * cs_ref[...], axis=1, keepdims=True)
    lse = m_ref[...] + jnp.log(s_ref[...])
    per_row = lse - (1.0 - EPS) * tl - (EPS / N) * ss
    out_ref[0, 0] = jnp.sum(per_row) / B


def _combine_tc(x, gathered, m, s, cs, interpret=False):
    out = pl.pallas_call(
        _combine_body,
        out_specs=pl.BlockSpec(memory_space=pltpu.SMEM),
        out_shape=jax.ShapeDtypeStruct((1, 1), jnp.float32),
        interpret=interpret,
    )(x, gathered, m, s, cs)
    return out[0, 0]


def _loss_tc(x, features, gathered, interpret=False):
    m, s, cs = _flash_tc(x, features, interpret=interpret)
    return _combine_tc(x, gathered, m, s, cs, interpret=interpret)


def kernel(inputs, targets, features):
    gathered = _gather_rows_sc(features, targets.astype(jnp.int32))
    m, s, cs = _flash_tc(inputs, features)
    return _combine_tc(inputs, gathered, m, s, cs)
